# baseline (device time: 8297 ns/iter reference)
import jax
import jax.numpy as jnp
from jax import lax
from jax.experimental import pallas as pl
from jax.experimental.pallas import tpu as pltpu

N_Z = 4
K = 8


def _topk_cols(data, k):
    cols = []
    for i in range(k):
        m = jnp.max(data, axis=1, keepdims=True)
        cols.append(m)
        if i < k - 1:
            data = jnp.where(data == m, -jnp.inf, data)
    return jnp.concatenate(cols, axis=1)


def _topk_rows_axis0(data, k):
    rows = []
    for i in range(k):
        m = jnp.max(data, axis=0, keepdims=True)
        rows.append(m)
        if i < k - 1:
            data = jnp.where(data == m, -jnp.inf, data)
    return jnp.concatenate(rows, axis=0)


def kernel(x):
    m, n = x.shape

    def body(x_ref, out_ref, cand_ref, send_sems, recv_sems):
        my_x = lax.axis_index("x")
        my_y = lax.axis_index("y")
        my_z = lax.axis_index("z")

        barrier_sem = pltpu.get_barrier_semaphore()
        for dz in range(1, N_Z):
            pl.semaphore_signal(
                barrier_sem,
                inc=1,
                device_id=(my_x, my_y, (my_z + dz) % N_Z),
                device_id_type=pl.DeviceIdType.MESH,
            )

        cand_ref[0, :, :] = _topk_cols(x_ref[:, :], K).T

        pl.semaphore_wait(barrier_sem, N_Z - 1)

        rdmas = []
        for dz in range(1, N_Z):
            rdma = pltpu.make_async_remote_copy(
                src_ref=cand_ref.at[0],
                dst_ref=cand_ref.at[dz],
                send_sem=send_sems.at[dz - 1],
                recv_sem=recv_sems.at[dz - 1],
                device_id=(my_x, my_y, (my_z + dz) % N_Z),
                device_id_type=pl.DeviceIdType.MESH,
            )
            rdma.start()
            rdmas.append(rdma)
        for rdma in rdmas:
            rdma.wait_recv()

        allc = cand_ref[...].reshape(N_Z * K, m)
        out_ref[:, :] = _topk_rows_axis0(allc, K).T

        for rdma in rdmas:
            rdma.wait_send()

    return pl.pallas_call(
        body,
        out_shape=jax.ShapeDtypeStruct((m, K), jnp.float32),
        in_specs=[pl.BlockSpec(memory_space=pltpu.VMEM)],
        out_specs=pl.BlockSpec(memory_space=pltpu.VMEM),
        scratch_shapes=[
            pltpu.VMEM((N_Z, K, m), jnp.float32),
            pltpu.SemaphoreType.DMA((N_Z - 1,)),
            pltpu.SemaphoreType.DMA((N_Z - 1,)),
        ],
        compiler_params=pltpu.CompilerParams(collective_id=0),
    )(x)
